# baseline (speedup 1.0000x reference)
import math
import functools

import jax
import jax.numpy as jnp
from jax.experimental import pallas as pl
from jax.experimental.pallas import tpu as pltpu

_N = 10000
_F = 128
_R = 20
_CUT = 5.0
_L = 3
_E = 262144
_NP = 10240
_RB = 256
_CB = 1024


def _mask_kernel(pr, pc, m_ref, cnt_ref):
    i = pl.program_id(0)
    j = pl.program_id(1)

    @pl.when((i == 0) & (j == 0))
    def _():
        cnt_ref[0, 0] = 0

    xr, yr, zr = pr[0, :], pr[1, :], pr[2, :]
    xc, yc, zc = pc[0, :], pc[1, :], pc[2, :]

    def _b(t):
        return t.astype(jnp.bfloat16).astype(jnp.float32)

    cross = (_b(xr)[:, None] * _b(xc)[None, :]
             + _b(yr)[:, None] * _b(yc)[None, :]
             + _b(zr)[:, None] * _b(zc)[None, :])
    sqr = xr * xr + yr * yr + zr * zr
    sqc = xc * xc + yc * yc + zc * zc
    d2 = sqr[:, None] + sqc[None, :] - 2.0 * cross
    rid = i * _RB + jax.lax.broadcasted_iota(jnp.int32, (_RB, _CB), 0)
    cid = j * _CB + jax.lax.broadcasted_iota(jnp.int32, (_RB, _CB), 1)
    m = (d2 < _CUT * _CUT) & (rid != cid) & (rid < _N) & (cid < _N)
    m_ref[...] = m.astype(jnp.int8)
    cnt_ref[0, 0] += jnp.sum(m.astype(jnp.int32))


def _radius_graph(pos):
    posT = jnp.zeros((8, _NP), jnp.float32).at[:3, :_N].set(pos.T)
    mask, cnt = pl.pallas_call(
        _mask_kernel,
        grid=(_NP // _RB, _NP // _CB),
        in_specs=[
            pl.BlockSpec((8, _RB), lambda i, j: (0, i)),
            pl.BlockSpec((8, _CB), lambda i, j: (0, j)),
        ],
        out_specs=[
            pl.BlockSpec((_RB, _CB), lambda i, j: (i, j)),
            pl.BlockSpec(memory_space=pltpu.SMEM),
        ],
        out_shape=[
            jax.ShapeDtypeStruct((_NP, _NP), jnp.int8),
            jax.ShapeDtypeStruct((1, 1), jnp.int32),
        ],
    )(posT, posT)
    dst, src = jnp.nonzero(mask, size=_E, fill_value=0)
    count = cnt[0, 0]
    valid = jnp.arange(_E) < count
    return dst.astype(jnp.int32), src.astype(jnp.int32), valid


def _silu(x):
    return x * jax.nn.sigmoid(x)


def _bessel(d):
    freqs = jnp.arange(1, _R + 1, dtype=jnp.float32) * math.pi / _CUT
    ax = d[:, None] * freqs[None, :]
    norm = jnp.where(d == 0, 1.0, d)
    return jnp.sin(ax) / norm[:, None]


def _cutoff_fn(d):
    return 0.5 * (jnp.cos(d * math.pi / _CUT) + 1.0) * (d < _CUT).astype(jnp.float32)


def kernel(z, pos, params):
    dst, src, valid = _radius_graph(pos)

    s = params["embedding"][z]
    v = jnp.zeros((_N, _F, 3), dtype=jnp.float32)
    rij = pos[dst] - pos[src]
    d = jnp.linalg.norm(rij, axis=1)
    rbf = _bessel(d)
    cut = _cutoff_fn(d)
    rn = rij / jnp.maximum(d[:, None], 1e-12)

    for L in range(_L):
        mp = params["msg"][L]
        up = params["upd"][L]

        phi_n = _silu(s @ mp["lin1"]["W"] + mp["lin1"]["b"])
        phi_n = phi_n @ mp["lin2"]["W"] + mp["lin2"]["b"]
        W = (rbf @ mp["lin_rbf"]["W"] + mp["lin_rbf"]["b"]) * cut[:, None]
        pw = phi_n[src] * W
        pw = jnp.where(valid[:, None], pw, 0.0)
        left = pw[:, :_F]
        dsm = pw[:, _F:2 * _F]
        right = pw[:, 2 * _F:]
        dvm = v[src] * left[:, :, None] + right[:, :, None] * rn[:, None, :]
        ds = jax.ops.segment_sum(dsm, dst, num_segments=_N)
        dv = jax.ops.segment_sum(dvm, dst, num_segments=_N)
        s = ds + s
        v = dv + v

        v_ut = jnp.swapaxes(v, 1, 2)
        U_v = jnp.swapaxes(v_ut @ up["denseU"]["W"], 1, 2)
        V_v = jnp.swapaxes(v_ut @ up["denseV"]["W"], 1, 2)
        dot = jnp.sum(U_v * V_v, axis=-1)
        V_norm = jnp.sqrt(jnp.sum(V_v * V_v, axis=-1) + 1e-12)
        a = jnp.concatenate([s, V_norm], axis=-1)
        a = _silu(a @ up["lin_up"]["W"] + up["lin_up"]["b"])
        a = a @ up["lin2"]["W"] + up["lin2"]["b"]
        a_vv = a[:, :_F]
        a_sv = a[:, _F:2 * _F]
        a_ss = a[:, 2 * _F:]
        s = s + a_ss + a_sv * dot
        v = v + U_v * a_vv[:, :, None]

    W = params["lin"]["W"]
    b = params["lin"]["b"]
    s = _silu(s @ W + b)
    s = s @ W + b
    return s
